# Initial kernel scaffold; baseline (speedup 1.0000x reference)
import jax
import jax.numpy as jnp
from jax.experimental import pallas as pl


def kernel(batch_pd, batch_gt):
    raise NotImplementedError("write your pallas kernel here")


# speedup vs baseline: 1.0664x; 1.0664x over previous
import functools
import math

import jax
import jax.numpy as jnp
from jax.experimental import pallas as pl
from jax.experimental.pallas import tpu as pltpu

_THRES = math.log(0.7 / (1 - 0.7))
_RADII = (0.74, 0.528)
_BATCH_PICK = 15


def _sig(x):
    return 1.0 / (1.0 + jnp.exp(-x))


def _body(tall_pd_ref, wide_pd_ref, tall_gt_ref, wide_gt_ref, out_ref,
          rows_pd_ref, rows_gt_ref, w_ref, res_ref, *,
          n, zsh, xsh, xmask, ymask, zup, si, sj, radii, thres):
    f32 = jnp.float32

    def pos_rows(wide_ref, rows_ref, e):
        fj = jax.lax.broadcasted_iota(jnp.int32, (1, n), 1)
        zj = (fj >> zsh).astype(f32) + _sig(wide_ref[e, 1:2, :])
        xj = ((fj >> xsh) & xmask).astype(f32) + _sig(wide_ref[e, 2:3, :])
        yj = (fj & ymask).astype(f32) + _sig(wide_ref[e, 3:4, :])
        cj = wide_ref[e, 0:1, :]
        rows_ref[0:1, :] = zj
        rows_ref[1:2, :] = xj
        rows_ref[2:3, :] = yj
        rows_ref[3:4, :] = cj
        return zj, xj, yj, cj

    def sweep(e, tall_ref, rows_ref, w_row, c2, use_ord):
        w_ref[0:1, :] = w_row
        ni = n // si
        nj = n // sj
        row0 = jax.lax.broadcasted_iota(jnp.int32, (8, si), 0) == 0

        def jbody(jb, carry):
            j0 = jb * sj
            zj = rows_ref[0:1, pl.ds(j0, sj)]
            xj = rows_ref[1:2, pl.ds(j0, sj)]
            yj = rows_ref[2:3, pl.ds(j0, sj)]
            cj = rows_ref[3:4, pl.ds(j0, sj)]
            fj = jax.lax.broadcasted_iota(jnp.int32, (1, sj), 1) + j0

            def ibody(ib, acc):
                i0 = ib * si
                blk = tall_ref[e, pl.ds(i0, si), :]
                fi = jax.lax.broadcasted_iota(jnp.int32, (si, 1), 0) + i0
                ci = blk[:, 0:1]
                zi = (fi >> zsh).astype(f32) + _sig(blk[:, 1:2])
                xi = ((fi >> xsh) & xmask).astype(f32) + _sig(blk[:, 2:3])
                yi = (fi & ymask).astype(f32) + _sig(blk[:, 3:4])
                dz = zi - zj
                dx = xi - xj
                dy = yi - yj
                d2 = dz * dz + dx * dx + dy * dy
                hit = d2 < c2
                if use_ord:
                    hit = hit & ((ci > cj) | ((ci == cj) & (fi < fj)))
                tile = hit.astype(f32)
                wi = w_ref[0:1, pl.ds(i0, si)]
                w8 = jnp.where(row0, jnp.broadcast_to(wi, (8, si)), 0.0)
                return acc + jnp.dot(w8, tile, preferred_element_type=f32)

            acc = jax.lax.fori_loop(0, ni, ibody, jnp.zeros((8, sj), f32))
            res_ref[0:1, pl.ds(j0, sj)] = acc[0:1, :]
            return carry

        jax.lax.fori_loop(0, nj, jbody, 0)
        return res_ref[0:1, :]

    for e in range(2):
        cutoff2 = (2.0 * radii[e]) ** 2
        rad2 = radii[e] ** 2
        zp, _, _, cp = pos_rows(wide_pd_ref, rows_pd_ref, e)
        zg, _, _, cg = pos_rows(wide_gt_ref, rows_gt_ref, e)

        m = (cp > thres).astype(f32)
        r1 = sweep(e, tall_pd_ref, rows_pd_ref, m, cutoff2, True)
        w2 = m * (r1 == 0.0).astype(f32)
        s1 = sweep(e, tall_pd_ref, rows_pd_ref, w2, cutoff2, True)
        m2 = m * (s1 == 0.0).astype(f32)
        r2 = sweep(e, tall_pd_ref, rows_pd_ref, m2, cutoff2, True)
        w4 = m2 * (r2 == 0.0).astype(f32)
        s2 = sweep(e, tall_pd_ref, rows_pd_ref, w4, cutoff2, True)
        pv = m2 * (s2 == 0.0).astype(f32)

        st = sweep(e, tall_pd_ref, rows_gt_ref, pv, rad2, False)

        gv = cg > 0.0
        layer_p = pv * ((zp >= 0.0) & (zp < zup)).astype(f32)
        layer_t = gv & (zg >= 0.0) & (zg < zup)
        tv = jnp.sum(layer_t.astype(f32))
        pvl = jnp.sum(layer_p)
        tpv = jnp.sum((layer_t & (st > 0.0)).astype(f32))
        fpv = pvl - tpv
        fnv = tv - tpv
        arv = jnp.where(tv == 0.0, 1.0, tpv / jnp.maximum(tv, 1.0))
        apv = jnp.where(pvl == 0.0, 0.0, tpv / jnp.maximum(pvl, 1.0))
        accv = ((pvl == tpv) & (tv == tpv)).astype(f32)
        row = jnp.concatenate(
            [v.reshape(1, 1) for v in (tv, pvl, tpv, fpv, fnv, arv, apv, accv)],
            axis=1)
        out_ref[e, :, :] = row


def _build_call(zdim, xdim, ydim, si, sj, zup):
    n = zdim * xdim * ydim
    zsh = int(math.log2(xdim * ydim))
    xsh = int(math.log2(ydim))
    body = functools.partial(
        _body, n=n, zsh=zsh, xsh=xsh, xmask=xdim - 1, ymask=ydim - 1,
        zup=zup, si=si, sj=sj, radii=_RADII, thres=_THRES)

    def call(grid_pd, grid_gt):
        g_pd = grid_pd.reshape(n, 2, 4)
        g_gt = grid_gt.reshape(n, 2, 4)
        tall_pd = g_pd.transpose(1, 0, 2)
        wide_pd = g_pd.transpose(1, 2, 0)
        tall_gt = g_gt.transpose(1, 0, 2)
        wide_gt = g_gt.transpose(1, 2, 0)
        return pl.pallas_call(
            body,
            out_shape=jax.ShapeDtypeStruct((2, 1, 8), jnp.float32),
            scratch_shapes=[
                pltpu.VMEM((8, n), jnp.float32),
                pltpu.VMEM((8, n), jnp.float32),
                pltpu.VMEM((1, n), jnp.float32),
                pltpu.VMEM((1, n), jnp.float32),
            ],
        )(tall_pd, wide_pd, tall_gt, wide_gt)

    return call


def kernel(batch_pd, batch_gt):
    call = _build_call(8, 32, 32, 256, 2048, 8.0)
    return call(batch_pd[_BATCH_PICK], batch_gt[_BATCH_PICK])


# speedup vs baseline: 5.7573x; 5.3989x over previous
import functools
import math

import jax
import jax.numpy as jnp
from jax import lax
from jax.experimental import pallas as pl
from jax.experimental.pallas import tpu as pltpu
from jax.experimental.pallas import tpu_sc as plsc

_THRES = math.log(0.7 / (1 - 0.7))
_RADII = (0.74, 0.528)
_BATCH_PICK = 15
_NEG = float("-inf")
_F = 8


def _build_sc(n, zsh, xsh, xmask, ymask, thres):
    nv = n // 16
    mesh = plsc.VectorSubcoreMesh(core_axis_name="c", subcore_axis_name="s")

    @functools.partial(
        pl.kernel,
        mesh=mesh,
        compiler_params=pltpu.CompilerParams(needs_layout_passes=False),
        out_type=[
            jax.ShapeDtypeStruct((4, n * _F), jnp.float32),
            jax.ShapeDtypeStruct((4, 16), jnp.int32),
        ],
        scratch_types=[
            pltpu.VMEM((n,), jnp.float32),
            pltpu.VMEM((n,), jnp.float32),
            pltpu.VMEM((n,), jnp.float32),
            pltpu.VMEM((n,), jnp.float32),
            pltpu.VMEM((n * _F,), jnp.float32),
            pltpu.VMEM((16,), jnp.int32),
        ],
    )
    def sc_kernel(pd_hbm, gt_hbm, feat_hbm, cnt_hbm,
                  conf_v, oz_v, ox_v, oy_v, out_v, cnt_v):
        wid = lax.axis_index("s") * 2 + lax.axis_index("c")
        f32 = jnp.float32
        neg = jnp.full((16,), _NEG, f32)
        iota = lax.iota(jnp.int32, 16)

        for pair in range(4):

            @pl.when(wid == pair)
            def _():
                src = pd_hbm if pair < 2 else gt_hbm
                e = pair % 2
                th = thres if pair < 2 else 0.0
                pltpu.sync_copy(src.at[e, 0], conf_v)
                pltpu.sync_copy(src.at[e, 1], oz_v)
                pltpu.sync_copy(src.at[e, 2], ox_v)
                pltpu.sync_copy(src.at[e, 3], oy_v)

                def initb(i, carry):
                    out_v[pl.ds(i * 16, 16)] = neg
                    return carry

                lax.fori_loop(0, n * _F // 16, initb, 0)
                cnt_v[...] = jnp.zeros((16,), jnp.int32)
                one = jnp.full((16,), 1, jnp.int32)
                zero = jnp.full((16,), 0, jnp.int32)

                def body(i, carry):
                    f = iota + i * 16
                    c = conf_v[pl.ds(i * 16, 16)]
                    m = c > th
                    z = (f >> zsh).astype(f32) + 1.0 / (
                        1.0 + jnp.exp(-oz_v[pl.ds(i * 16, 16)]))
                    x = ((f >> xsh) & xmask).astype(f32) + 1.0 / (
                        1.0 + jnp.exp(-ox_v[pl.ds(i * 16, 16)]))
                    y = (f & ymask).astype(f32) + 1.0 / (
                        1.0 + jnp.exp(-oy_v[pl.ds(i * 16, 16)]))
                    mi = jnp.where(m, one, zero)
                    cnt = cnt_v[...]
                    b8 = (cnt + plsc.cumsum(mi) - mi) * _F
                    plsc.store_scatter(out_v, [b8 + 0], z, mask=m)
                    plsc.store_scatter(out_v, [b8 + 1], x, mask=m)
                    plsc.store_scatter(out_v, [b8 + 2], y, mask=m)
                    plsc.store_scatter(out_v, [b8 + 3], c, mask=m)
                    plsc.store_scatter(out_v, [b8 + 4], f.astype(f32), mask=m)
                    cnt_v[...] = cnt + plsc.all_reduce_population_count(m)
                    return carry

                lax.fori_loop(0, nv, body, 0)
                pltpu.sync_copy(out_v, feat_hbm.at[pair])
                pltpu.sync_copy(cnt_v, cnt_hbm.at[pair])

    return sc_kernel


def _tc_body(cnt_ref, pd_tall_ref, pd_wide_ref, gt_wide_ref, out_ref,
             w_ref, res_ref, *, n, zup, si, sj, radii, thres):
    f32 = jnp.float32

    def cdiv(a, b):
        return (a + b - 1) // b

    def sweep(e, wide_ref, w_row, c2, use_ord, ni, nj):
        w_ref[0:1, :] = w_row
        res_ref[0:1, :] = jnp.zeros((1, n), f32)
        row0 = lax.broadcasted_iota(jnp.int32, (8, si), 0) == 0

        def jbody(jb, carry):
            j0 = jb * sj
            zj = wide_ref[e, 0:1, pl.ds(j0, sj)]
            xj = wide_ref[e, 1:2, pl.ds(j0, sj)]
            yj = wide_ref[e, 2:3, pl.ds(j0, sj)]
            cj = wide_ref[e, 3:4, pl.ds(j0, sj)]
            fj = wide_ref[e, 4:5, pl.ds(j0, sj)]

            def ibody(ib, acc):
                i0 = ib * si
                blk = pd_tall_ref[e, pl.ds(i0, si), :]
                zi = blk[:, 0:1]
                xi = blk[:, 1:2]
                yi = blk[:, 2:3]
                ci = blk[:, 3:4]
                fi = blk[:, 4:5]
                dz = zi - zj
                dx = xi - xj
                dy = yi - yj
                d2 = dz * dz + dx * dx + dy * dy
                hit = d2 < c2
                if use_ord:
                    hit = hit & ((ci > cj) | ((ci == cj) & (fi < fj)))
                tile = hit.astype(f32)
                wi = w_ref[0:1, pl.ds(i0, si)]
                w8 = jnp.where(row0, jnp.broadcast_to(wi, (8, si)), 0.0)
                return acc + jnp.dot(w8, tile, preferred_element_type=f32)

            acc = lax.fori_loop(0, ni, ibody, jnp.zeros((8, sj), f32))
            res_ref[0:1, pl.ds(j0, sj)] = acc[0:1, :]
            return carry

        lax.fori_loop(0, nj, jbody, 0)
        return res_ref[0:1, :]

    for e in range(2):
        cutoff2 = (2.0 * radii[e]) ** 2
        rad2 = radii[e] ** 2
        vp = cnt_ref[e]
        vg = cnt_ref[2 + e]
        nip = cdiv(vp, si)
        njp = cdiv(vp, sj)
        njg = cdiv(vg, sj)

        zp = pd_wide_ref[e, 0:1, :]
        cp = pd_wide_ref[e, 3:4, :]
        zg = gt_wide_ref[e, 0:1, :]
        cg = gt_wide_ref[e, 3:4, :]

        m = (cp > thres).astype(f32)
        r1 = sweep(e, pd_wide_ref, m, cutoff2, True, nip, njp)
        w2 = m * (r1 == 0.0).astype(f32)
        s1 = sweep(e, pd_wide_ref, w2, cutoff2, True, nip, njp)
        m2 = m * (s1 == 0.0).astype(f32)
        r2 = sweep(e, pd_wide_ref, m2, cutoff2, True, nip, njp)
        w4 = m2 * (r2 == 0.0).astype(f32)
        s2 = sweep(e, pd_wide_ref, w4, cutoff2, True, nip, njp)
        pv = m2 * (s2 == 0.0).astype(f32)

        st = sweep(e, gt_wide_ref, pv, rad2, False, nip, njg)

        gv = cg > 0.0
        layer_p = pv * ((zp >= 0.0) & (zp < zup)).astype(f32)
        layer_t = gv & (zg >= 0.0) & (zg < zup)
        tv = jnp.sum(layer_t.astype(f32))
        pvl = jnp.sum(layer_p)
        tpv = jnp.sum((layer_t & (st > 0.0)).astype(f32))
        fpv = pvl - tpv
        fnv = tv - tpv
        arv = jnp.where(tv == 0.0, 1.0, tpv / jnp.maximum(tv, 1.0))
        apv = jnp.where(pvl == 0.0, 0.0, tpv / jnp.maximum(pvl, 1.0))
        accv = ((pvl == tpv) & (tv == tpv)).astype(f32)
        row = jnp.concatenate(
            [v.reshape(1, 1) for v in (tv, pvl, tpv, fpv, fnv, arv, apv, accv)],
            axis=1)
        out_ref[e, :, :] = row


def _build_tc(n, zup, si, sj):
    body = functools.partial(_tc_body, n=n, zup=zup, si=si, sj=sj,
                             radii=_RADII, thres=_THRES)

    def call(cnt4, pd_tall, pd_wide, gt_wide):
        return pl.pallas_call(
            body,
            out_shape=jax.ShapeDtypeStruct((2, 1, 8), jnp.float32),
            in_specs=[
                pl.BlockSpec(memory_space=pltpu.SMEM),
                pl.BlockSpec(memory_space=pltpu.VMEM),
                pl.BlockSpec(memory_space=pltpu.VMEM),
                pl.BlockSpec(memory_space=pltpu.VMEM),
            ],
            scratch_shapes=[
                pltpu.VMEM((1, n), jnp.float32),
                pltpu.VMEM((1, n), jnp.float32),
            ],
        )(cnt4, pd_tall, pd_wide, gt_wide)

    return call


def _analyse(grid_pd, grid_gt, zdim, xdim, ydim, si, sj, zup):
    n = zdim * xdim * ydim
    zsh = int(math.log2(xdim * ydim))
    xsh = int(math.log2(ydim))
    wide_pd = grid_pd.reshape(n, 2, 4).transpose(1, 2, 0)
    wide_gt = grid_gt.reshape(n, 2, 4).transpose(1, 2, 0)

    sc = _build_sc(n, zsh, xsh, xdim - 1, ydim - 1, _THRES)
    feats, cnts = sc(wide_pd, wide_gt)

    cnt4 = cnts[:, 0]
    tall = feats.reshape(4, n, _F)
    pd_tall = tall[0:2]
    pd_wide = pd_tall.transpose(0, 2, 1)
    gt_wide = tall[2:4].transpose(0, 2, 1)
    tc = _build_tc(n, zup, si, sj)
    return tc(cnt4, pd_tall, pd_wide, gt_wide)


def kernel(batch_pd, batch_gt):
    return _analyse(batch_pd[_BATCH_PICK], batch_gt[_BATCH_PICK],
                    8, 32, 32, 256, 512, 8.0)


# speedup vs baseline: 7.3407x; 1.2750x over previous
import functools
import math

import jax
import jax.numpy as jnp
from jax import lax
from jax.experimental import pallas as pl
from jax.experimental.pallas import tpu as pltpu
from jax.experimental.pallas import tpu_sc as plsc

_THRES = math.log(0.7 / (1 - 0.7))
_RADII = (0.74, 0.528)
_BATCH_PICK = 15
_NEG = float("-inf")
_F = 8


def _build_sc(n, zsh, xsh, xmask, ymask, thres):
    nv = n // 16
    mesh = plsc.VectorSubcoreMesh(core_axis_name="c", subcore_axis_name="s")

    @functools.partial(
        pl.kernel,
        mesh=mesh,
        compiler_params=pltpu.CompilerParams(needs_layout_passes=False),
        out_type=[
            jax.ShapeDtypeStruct((4, n * _F), jnp.float32),
            jax.ShapeDtypeStruct((4, 16), jnp.int32),
        ],
        scratch_types=[
            pltpu.VMEM((n,), jnp.float32),
            pltpu.VMEM((n,), jnp.float32),
            pltpu.VMEM((n,), jnp.float32),
            pltpu.VMEM((n,), jnp.float32),
            pltpu.VMEM((n * _F,), jnp.float32),
            pltpu.VMEM((16,), jnp.int32),
        ],
    )
    def sc_kernel(pd_hbm, gt_hbm, feat_hbm, cnt_hbm,
                  conf_v, oz_v, ox_v, oy_v, out_v, cnt_v):
        wid = lax.axis_index("s") * 2 + lax.axis_index("c")
        f32 = jnp.float32
        neg = jnp.full((16,), _NEG, f32)
        iota = lax.iota(jnp.int32, 16)

        for pair in range(4):

            @pl.when(wid == pair)
            def _():
                src = pd_hbm if pair < 2 else gt_hbm
                e = pair % 2
                th = thres if pair < 2 else 0.0
                pltpu.sync_copy(src.at[e, 0], conf_v)
                pltpu.sync_copy(src.at[e, 1], oz_v)
                pltpu.sync_copy(src.at[e, 2], ox_v)
                pltpu.sync_copy(src.at[e, 3], oy_v)

                def initb(i, carry):
                    out_v[pl.ds(i * 16, 16)] = neg
                    return carry

                lax.fori_loop(0, n * _F // 16, initb, 0)
                cnt_v[...] = jnp.zeros((16,), jnp.int32)
                one = jnp.full((16,), 1, jnp.int32)
                zero = jnp.full((16,), 0, jnp.int32)

                def body(i, carry):
                    f = iota + i * 16
                    c = conf_v[pl.ds(i * 16, 16)]
                    m = c > th
                    z = (f >> zsh).astype(f32) + 1.0 / (
                        1.0 + jnp.exp(-oz_v[pl.ds(i * 16, 16)]))
                    x = ((f >> xsh) & xmask).astype(f32) + 1.0 / (
                        1.0 + jnp.exp(-ox_v[pl.ds(i * 16, 16)]))
                    y = (f & ymask).astype(f32) + 1.0 / (
                        1.0 + jnp.exp(-oy_v[pl.ds(i * 16, 16)]))
                    mi = jnp.where(m, one, zero)
                    cnt = cnt_v[...]
                    b8 = (cnt + plsc.cumsum(mi) - mi) * _F
                    plsc.store_scatter(out_v, [b8 + 0], z, mask=m)
                    plsc.store_scatter(out_v, [b8 + 1], x, mask=m)
                    plsc.store_scatter(out_v, [b8 + 2], y, mask=m)
                    plsc.store_scatter(out_v, [b8 + 3], c, mask=m)
                    plsc.store_scatter(out_v, [b8 + 4], f.astype(f32), mask=m)
                    cnt_v[...] = cnt + plsc.all_reduce_population_count(m)
                    return carry

                lax.fori_loop(0, nv, body, 0)
                pltpu.sync_copy(out_v, feat_hbm.at[pair])
                pltpu.sync_copy(cnt_v, cnt_hbm.at[pair])

    return sc_kernel


def _tc_body(cnt_ref, pd_tall_ref, pd_wide_ref, gt_wide_ref, out_ref,
             w_ref, res_ref, tcache_ref, *, n, zup, si, sj, cap, radii,
             thres):
    f32 = jnp.float32
    bf16 = jnp.bfloat16
    ncap = cap // si
    njcap = cap // sj

    def cdiv(a, b):
        return (a + b - 1) // b

    row0 = lax.broadcasted_iota(jnp.int32, (8, si), 0) == 0

    def w8_of(i0):
        wi = w_ref[0:1, pl.ds(i0, si)]
        return jnp.where(row0, jnp.broadcast_to(wi, (8, si)), 0.0)

    def sweep(e, wide_ref, w_row, c2, use_ord, ni, nj, store_cache=False):
        w_ref[0:1, :] = w_row
        res_ref[0:1, :] = jnp.zeros((1, n), f32)

        def jbody(jb, carry):
            j0 = jb * sj
            zj = wide_ref[e, 0:1, pl.ds(j0, sj)]
            xj = wide_ref[e, 1:2, pl.ds(j0, sj)]
            yj = wide_ref[e, 2:3, pl.ds(j0, sj)]
            cj = wide_ref[e, 3:4, pl.ds(j0, sj)]
            fj = wide_ref[e, 4:5, pl.ds(j0, sj)]

            def ibody(ib, acc):
                i0 = ib * si
                blk = pd_tall_ref[e, pl.ds(i0, si), :]
                zi = blk[:, 0:1]
                xi = blk[:, 1:2]
                yi = blk[:, 2:3]
                ci = blk[:, 3:4]
                fi = blk[:, 4:5]
                dz = zi - zj
                dx = xi - xj
                dy = yi - yj
                d2 = dz * dz + dx * dx + dy * dy
                hit = d2 < c2
                if use_ord:
                    hit = hit & ((ci > cj) | ((ci == cj) & (fi < fj)))
                tile = hit.astype(f32)
                if store_cache:
                    @pl.when((i0 + si <= cap) & (j0 + sj <= cap))
                    def _():
                        tcache_ref[pl.ds(i0, si), pl.ds(j0, sj)] = (
                            tile.astype(bf16))
                return acc + jnp.dot(w8_of(i0), tile,
                                     preferred_element_type=f32)

            acc = lax.fori_loop(0, ni, ibody, jnp.zeros((8, sj), f32))
            res_ref[0:1, pl.ds(j0, sj)] = acc[0:1, :]
            return carry

        lax.fori_loop(0, nj, jbody, 0)
        return res_ref[0:1, :]

    def sweep_cached(w_row, ni, nj):
        w_ref[0:1, :] = w_row
        res_ref[0:1, :] = jnp.zeros((1, n), f32)

        def jbody(jb, carry):
            j0 = jb * sj

            def ibody(ib, acc):
                i0 = ib * si
                tblk = tcache_ref[pl.ds(i0, si), pl.ds(j0, sj)]
                return acc + jnp.dot(w8_of(i0).astype(bf16), tblk,
                                     preferred_element_type=f32)

            acc = lax.fori_loop(0, ni, ibody, jnp.zeros((8, sj), f32))
            res_ref[0:1, pl.ds(j0, sj)] = acc[0:1, :]
            return carry

        lax.fori_loop(0, nj, jbody, 0)
        return res_ref[0:1, :]

    def sweep_nms(e, fits, w_row, c2, ni, nj):
        @pl.when(fits)
        def _():
            sweep_cached(w_row, ni, nj)

        @pl.when(jnp.logical_not(fits))
        def _():
            sweep(e, pd_wide_ref, w_row, c2, True, ni, nj)

        return res_ref[0:1, :]

    for e in range(2):
        cutoff2 = (2.0 * radii[e]) ** 2
        rad2 = radii[e] ** 2
        vp = cnt_ref[e]
        vg = cnt_ref[2 + e]
        nip = cdiv(vp, si)
        njp = cdiv(vp, sj)
        njg = cdiv(vg, sj)

        zp = pd_wide_ref[e, 0:1, :]
        cp = pd_wide_ref[e, 3:4, :]
        zg = gt_wide_ref[e, 0:1, :]
        cg = gt_wide_ref[e, 3:4, :]

        fits = vp <= cap
        m = (cp > thres).astype(f32)
        r1 = sweep(e, pd_wide_ref, m, cutoff2, True, nip, njp,
                   store_cache=True)
        w2 = m * (r1 == 0.0).astype(f32)
        s1 = sweep_nms(e, fits, w2, cutoff2, nip, njp)
        m2 = m * (s1 == 0.0).astype(f32)
        r2 = sweep_nms(e, fits, m2, cutoff2, nip, njp)
        w4 = m2 * (r2 == 0.0).astype(f32)
        s2 = sweep_nms(e, fits, w4, cutoff2, nip, njp)
        pv = m2 * (s2 == 0.0).astype(f32)

        st = sweep(e, gt_wide_ref, pv, rad2, False, nip, njg)

        gv = cg > 0.0
        layer_p = pv * ((zp >= 0.0) & (zp < zup)).astype(f32)
        layer_t = gv & (zg >= 0.0) & (zg < zup)
        tv = jnp.sum(layer_t.astype(f32))
        pvl = jnp.sum(layer_p)
        tpv = jnp.sum((layer_t & (st > 0.0)).astype(f32))
        fpv = pvl - tpv
        fnv = tv - tpv
        arv = jnp.where(tv == 0.0, 1.0, tpv / jnp.maximum(tv, 1.0))
        apv = jnp.where(pvl == 0.0, 0.0, tpv / jnp.maximum(pvl, 1.0))
        accv = ((pvl == tpv) & (tv == tpv)).astype(f32)
        row = jnp.concatenate(
            [v.reshape(1, 1) for v in (tv, pvl, tpv, fpv, fnv, arv, apv, accv)],
            axis=1)
        out_ref[e, :, :] = row


def _build_tc(n, zup, si, sj, cap):
    body = functools.partial(_tc_body, n=n, zup=zup, si=si, sj=sj, cap=cap,
                             radii=_RADII, thres=_THRES)

    def call(cnt4, pd_tall, pd_wide, gt_wide):
        return pl.pallas_call(
            body,
            out_shape=jax.ShapeDtypeStruct((2, 1, 8), jnp.float32),
            in_specs=[
                pl.BlockSpec(memory_space=pltpu.SMEM),
                pl.BlockSpec(memory_space=pltpu.VMEM),
                pl.BlockSpec(memory_space=pltpu.VMEM),
                pl.BlockSpec(memory_space=pltpu.VMEM),
            ],
            scratch_shapes=[
                pltpu.VMEM((1, n), jnp.float32),
                pltpu.VMEM((1, n), jnp.float32),
                pltpu.VMEM((cap, cap), jnp.bfloat16),
            ],
        )(cnt4, pd_tall, pd_wide, gt_wide)

    return call


def _analyse(grid_pd, grid_gt, zdim, xdim, ydim, si, sj, zup, cap):
    n = zdim * xdim * ydim
    zsh = int(math.log2(xdim * ydim))
    xsh = int(math.log2(ydim))
    wide_pd = grid_pd.reshape(n, 2, 4).transpose(1, 2, 0)
    wide_gt = grid_gt.reshape(n, 2, 4).transpose(1, 2, 0)

    sc = _build_sc(n, zsh, xsh, xdim - 1, ydim - 1, _THRES)
    feats, cnts = sc(wide_pd, wide_gt)

    cnt4 = cnts[:, 0]
    tall = feats.reshape(4, n, _F)
    pd_tall = tall[0:2]
    pd_wide = pd_tall.transpose(0, 2, 1)
    gt_wide = tall[2:4].transpose(0, 2, 1)
    tc = _build_tc(n, zup, si, sj, cap)
    return tc(cnt4, pd_tall, pd_wide, gt_wide)


def kernel(batch_pd, batch_gt):
    return _analyse(batch_pd[_BATCH_PICK], batch_gt[_BATCH_PICK],
                    8, 32, 32, 256, 512, 8.0, 2048)


# speedup vs baseline: 9.9767x; 1.3591x over previous
import functools
import math

import jax
import jax.numpy as jnp
from jax import lax
from jax.experimental import pallas as pl
from jax.experimental.pallas import tpu as pltpu
from jax.experimental.pallas import tpu_sc as plsc

_THRES = math.log(0.7 / (1 - 0.7))
_RADII = (0.74, 0.528)
_BATCH_PICK = 15
_NEG = float("-inf")
_F = 8


def _build_sc(n, zsh, xsh, xmask, ymask, thres):
    ns = n // 8
    nv = ns // 16
    ck = ns * _F
    mesh = plsc.VectorSubcoreMesh(core_axis_name="c", subcore_axis_name="s")

    @functools.partial(
        pl.kernel,
        mesh=mesh,
        compiler_params=pltpu.CompilerParams(needs_layout_passes=False),
        out_type=[
            jax.ShapeDtypeStruct((4, n * _F), jnp.float32),
            jax.ShapeDtypeStruct((4, 16), jnp.int32),
        ],
        scratch_types=[
            pltpu.VMEM((ns,), jnp.float32),
            pltpu.VMEM((ns,), jnp.float32),
            pltpu.VMEM((ns,), jnp.float32),
            pltpu.VMEM((ns,), jnp.float32),
            pltpu.VMEM((_F, ns), jnp.float32),
            pltpu.VMEM((n * _F,), jnp.float32),
            pltpu.VMEM((16,), jnp.int32),
            pltpu.VMEM((8 * 16,), jnp.int32),
            pltpu.VMEM_SHARED((2 * 8 * ck,), jnp.float32),
            pltpu.VMEM_SHARED((2 * 8 * 16,), jnp.int32),
            pltpu.SemaphoreType.DMA,
        ],
    )
    def sc_kernel(pd_hbm, gt_hbm, feat_hbm, cnt_hbm,
                  conf_v, oz_v, ox_v, oy_v, chunk_v, final_v, cnt_v,
                  cnts8_v, spm_feat, spm_cnt, sem):
        cid = lax.axis_index("c")
        sid = lax.axis_index("s")
        s = sid % 8
        f32 = jnp.float32
        neg = jnp.full((16,), _NEG, f32)
        zf = jnp.zeros((16,), f32)
        one = jnp.full((16,), 1, jnp.int32)
        zero = jnp.full((16,), 0, jnp.int32)
        iota = lax.iota(jnp.int32, 16)
        base = s * ns

        def on_pair(p):
            return (cid == p // 2) & (sid // 8 == p % 2)

        for p in range(4):

            @pl.when(on_pair(p))
            def _(p=p):
                src = pd_hbm if p < 2 else gt_hbm
                e = p % 2
                th = thres if p < 2 else 0.0
                pltpu.sync_copy(src.at[e, 0, pl.ds(base, ns)], conf_v)
                pltpu.sync_copy(src.at[e, 1, pl.ds(base, ns)], oz_v)
                pltpu.sync_copy(src.at[e, 2, pl.ds(base, ns)], ox_v)
                pltpu.sync_copy(src.at[e, 3, pl.ds(base, ns)], oy_v)

                def initb(i, carry):
                    for c in range(_F):
                        chunk_v[c, pl.ds(i * 16, 16)] = neg if c < 5 else zf
                    return carry

                lax.fori_loop(0, ns // 16, initb, 0)
                cnt_v[...] = jnp.zeros((16,), jnp.int32)

                def body(i, carry):
                    f = iota + (base + i * 16)
                    c = conf_v[pl.ds(i * 16, 16)]
                    m = c > th
                    z = (f >> zsh).astype(f32) + 1.0 / (
                        1.0 + jnp.exp(-oz_v[pl.ds(i * 16, 16)]))
                    x = ((f >> xsh) & xmask).astype(f32) + 1.0 / (
                        1.0 + jnp.exp(-ox_v[pl.ds(i * 16, 16)]))
                    y = (f & ymask).astype(f32) + 1.0 / (
                        1.0 + jnp.exp(-oy_v[pl.ds(i * 16, 16)]))
                    mi = jnp.where(m, one, zero)
                    cnt = cnt_v[...]
                    t = cnt + plsc.cumsum(mi) - mi
                    for cc, vals in ((0, z), (1, x), (2, y), (3, c),
                                     (4, f.astype(f32))):
                        plsc.store_scatter(
                            chunk_v, [jnp.full((16,), cc, jnp.int32), t],
                            vals, mask=m)
                    cnt_v[...] = cnt + plsc.all_reduce_population_count(m)
                    return carry

                lax.fori_loop(0, nv, body, 0)
                cnt_v[...] = (cnt_v[...] + 7) & ~7
                wbase = (p % 2) * 8 * ck
                for c in range(_F):
                    pltpu.sync_copy(
                        chunk_v.at[c],
                        spm_feat.at[pl.ds(wbase + s * ck + c * ns, ns)])
                pltpu.sync_copy(
                    cnt_v,
                    spm_cnt.at[pl.ds((p % 2) * 8 * 16 + s * 16, 16)])

                @pl.when(s == 0)
                def _():
                    def initf(i, carry):
                        final_v[pl.ds(3 * n + i * 16, 16)] = neg
                        return carry

                    lax.fori_loop(0, n // 16, initf, 0)

        plsc.subcore_barrier()

        for p in range(4):

            @pl.when(on_pair(p) & (s == 0))
            def _(p=p):
                sbase = (p % 2) * 8
                pltpu.sync_copy(spm_cnt.at[pl.ds(sbase * 16, 8 * 16)],
                                cnts8_v)
                off = jnp.int32(0)
                for k in range(8):
                    offm = pl.multiple_of(off, 8)
                    handles = [pltpu.async_copy(
                        spm_feat.at[pl.ds((sbase + k) * ck + c * ns, ns)],
                        final_v.at[pl.ds(c * n + offm, ns)], sem)
                        for c in range(8)]
                    for h in handles:
                        h.wait()
                    off = off + cnts8_v[pl.ds(k * 16, 16)][0]
                cnt_v[...] = zero + off
                pltpu.sync_copy(final_v, feat_hbm.at[p])
                pltpu.sync_copy(cnt_v, cnt_hbm.at[p])

    return sc_kernel


def _tc_body(cnt_ref, pd_tall_ref, pd_wide_ref, gt_wide_ref, out_ref,
             w_ref, res_ref, tcache_ref, *, n, zup, si, sj, cap, radii,
             thres):
    f32 = jnp.float32
    bf16 = jnp.bfloat16
    ncap = cap // si
    njcap = cap // sj

    def cdiv(a, b):
        return (a + b - 1) // b

    row0 = lax.broadcasted_iota(jnp.int32, (8, si), 0) == 0

    def w8_of(i0):
        wi = w_ref[0:1, pl.ds(i0, si)]
        return jnp.where(row0, jnp.broadcast_to(wi, (8, si)), 0.0)

    def sweep(e, wide_ref, w_row, c2, use_ord, ni, nj, store_cache=False):
        w_ref[0:1, :] = w_row
        res_ref[0:1, :] = jnp.zeros((1, n), f32)

        def jbody(jb, carry):
            j0 = jb * sj
            zj = wide_ref[e, 0:1, pl.ds(j0, sj)]
            xj = wide_ref[e, 1:2, pl.ds(j0, sj)]
            yj = wide_ref[e, 2:3, pl.ds(j0, sj)]
            cj = wide_ref[e, 3:4, pl.ds(j0, sj)]
            fj = wide_ref[e, 4:5, pl.ds(j0, sj)]

            def ibody(ib, acc):
                i0 = ib * si
                blk = pd_tall_ref[e, pl.ds(i0, si), :]
                zi = blk[:, 0:1]
                xi = blk[:, 1:2]
                yi = blk[:, 2:3]
                ci = blk[:, 3:4]
                fi = blk[:, 4:5]
                dz = zi - zj
                dx = xi - xj
                dy = yi - yj
                d2 = dz * dz + dx * dx + dy * dy
                hit = d2 < c2
                if use_ord:
                    hit = hit & ((ci > cj) | ((ci == cj) & (fi < fj)))
                tile = hit.astype(f32)
                if store_cache:
                    @pl.when((i0 + si <= cap) & (j0 + sj <= cap))
                    def _():
                        tcache_ref[pl.ds(i0, si), pl.ds(j0, sj)] = (
                            tile.astype(bf16))
                return acc + jnp.dot(w8_of(i0), tile,
                                     preferred_element_type=f32)

            acc = lax.fori_loop(0, ni, ibody, jnp.zeros((8, sj), f32))
            res_ref[0:1, pl.ds(j0, sj)] = acc[0:1, :]
            return carry

        lax.fori_loop(0, nj, jbody, 0)
        return res_ref[0:1, :]

    def sweep_cached(w_row, ni, nj):
        w_ref[0:1, :] = w_row
        res_ref[0:1, :] = jnp.zeros((1, n), f32)

        def jbody(jb, carry):
            j0 = jb * sj

            def ibody(ib, acc):
                i0 = ib * si
                tblk = tcache_ref[pl.ds(i0, si), pl.ds(j0, sj)]
                return acc + jnp.dot(w8_of(i0).astype(bf16), tblk,
                                     preferred_element_type=f32)

            acc = lax.fori_loop(0, ni, ibody, jnp.zeros((8, sj), f32))
            res_ref[0:1, pl.ds(j0, sj)] = acc[0:1, :]
            return carry

        lax.fori_loop(0, nj, jbody, 0)
        return res_ref[0:1, :]

    def sweep_nms(e, fits, w_row, c2, ni, nj):
        @pl.when(fits)
        def _():
            sweep_cached(w_row, ni, nj)

        @pl.when(jnp.logical_not(fits))
        def _():
            sweep(e, pd_wide_ref, w_row, c2, True, ni, nj)

        return res_ref[0:1, :]

    for e in range(2):
        cutoff2 = (2.0 * radii[e]) ** 2
        rad2 = radii[e] ** 2
        vp = cnt_ref[e, 0]
        vg = cnt_ref[2 + e, 0]
        nip = cdiv(vp, si)
        njp = cdiv(vp, sj)
        njg = cdiv(vg, sj)

        zp = pd_wide_ref[e, 0:1, :]
        cp = pd_wide_ref[e, 3:4, :]
        zg = gt_wide_ref[e, 0:1, :]
        cg = gt_wide_ref[e, 3:4, :]

        fits = vp <= cap
        m = (cp > thres).astype(f32)
        r1 = sweep(e, pd_wide_ref, m, cutoff2, True, nip, njp,
                   store_cache=True)
        w2 = m * (r1 == 0.0).astype(f32)
        s1 = sweep_nms(e, fits, w2, cutoff2, nip, njp)
        m2 = m * (s1 == 0.0).astype(f32)
        r2 = sweep_nms(e, fits, m2, cutoff2, nip, njp)
        w4 = m2 * (r2 == 0.0).astype(f32)
        s2 = sweep_nms(e, fits, w4, cutoff2, nip, njp)
        pv = m2 * (s2 == 0.0).astype(f32)

        st = sweep(e, gt_wide_ref, pv, rad2, False, nip, njg)

        gv = cg > 0.0
        layer_p = pv * ((zp >= 0.0) & (zp < zup)).astype(f32)
        layer_t = gv & (zg >= 0.0) & (zg < zup)
        tv = jnp.sum(layer_t.astype(f32))
        pvl = jnp.sum(layer_p)
        tpv = jnp.sum((layer_t & (st > 0.0)).astype(f32))
        fpv = pvl - tpv
        fnv = tv - tpv
        arv = jnp.where(tv == 0.0, 1.0, tpv / jnp.maximum(tv, 1.0))
        apv = jnp.where(pvl == 0.0, 0.0, tpv / jnp.maximum(pvl, 1.0))
        accv = ((pvl == tpv) & (tv == tpv)).astype(f32)
        row = jnp.concatenate(
            [v.reshape(1, 1) for v in (tv, pvl, tpv, fpv, fnv, arv, apv, accv)],
            axis=1)
        out_ref[e, :, :] = row


def _build_tc(n, zup, si, sj, cap):
    body = functools.partial(_tc_body, n=n, zup=zup, si=si, sj=sj, cap=cap,
                             radii=_RADII, thres=_THRES)

    def call(cnts, pd_tall, pd_wide, gt_wide):
        return pl.pallas_call(
            body,
            out_shape=jax.ShapeDtypeStruct((2, 1, 8), jnp.float32),
            in_specs=[
                pl.BlockSpec(memory_space=pltpu.SMEM),
                pl.BlockSpec(memory_space=pltpu.VMEM),
                pl.BlockSpec(memory_space=pltpu.VMEM),
                pl.BlockSpec(memory_space=pltpu.VMEM),
            ],
            scratch_shapes=[
                pltpu.VMEM((1, n), jnp.float32),
                pltpu.VMEM((1, n), jnp.float32),
                pltpu.VMEM((cap, cap), jnp.bfloat16),
            ],
        )(cnts, pd_tall, pd_wide, gt_wide)

    return call


def _analyse(grid_pd, grid_gt, zdim, xdim, ydim, si, sj, zup, cap):
    n = zdim * xdim * ydim
    zsh = int(math.log2(xdim * ydim))
    xsh = int(math.log2(ydim))
    wide_pd = grid_pd.reshape(n, 2, 4).transpose(1, 2, 0)
    wide_gt = grid_gt.reshape(n, 2, 4).transpose(1, 2, 0)

    sc = _build_sc(n, zsh, xsh, xdim - 1, ydim - 1, _THRES)
    feats, cnts = sc(wide_pd, wide_gt)

    wide_all = feats.reshape(4, _F, n)
    pd_tall = wide_all[0:2].transpose(0, 2, 1)
    tc = _build_tc(n, zup, si, sj, cap)
    return tc(cnts, pd_tall, wide_all[0:2], wide_all[2:4])


def kernel(batch_pd, batch_gt):
    return _analyse(batch_pd[_BATCH_PICK], batch_gt[_BATCH_PICK],
                    8, 32, 32, 256, 512, 8.0, 2048)
